# Initial kernel scaffold; baseline (speedup 1.0000x reference)
import functools

import jax
import jax.numpy as jnp
from jax import lax
from jax.experimental import pallas as pl
from jax.experimental.pallas import tpu as pltpu
from jax.experimental.pallas import tpu_sc as plsc

N = 10000
E = 320000
H = 128
L = 5
OUT = 2
BN_EPS = 1e-5

NC = 2
NS = 16
NW = NC * NS
EDGES_PER_W = E // NW
CHUNK = 80
NCHUNK = EDGES_PER_W // CHUNK
RING = 5
ZROWS = 125
ROWS_PER_SUB = N // NS

NPAD = 10240
EMB_PER_W = NPAD // NW
EMB_CHUNKS = EMB_PER_W // CHUNK

_MESH = plsc.VectorSubcoreMesh(core_axis_name="c", subcore_axis_name="s")


@functools.partial(
    pl.kernel,
    out_type=jax.ShapeDtypeStruct((NC, N, H), jnp.float32),
    mesh=_MESH,
    scratch_types=[
        pltpu.VMEM((NCHUNK, CHUNK), jnp.int32),
        pltpu.VMEM((NCHUNK, CHUNK), jnp.int32),
        pltpu.VMEM((RING, CHUNK, H), jnp.float32),
        pltpu.VMEM((ZROWS, H), jnp.float32),
        pltpu.VMEM_SHARED((N, H), jnp.float32),
        pltpu.SemaphoreType.DMA,
        pltpu.SemaphoreType.DMA,
        pltpu.SemaphoreType.DMA,
        pltpu.SemaphoreType.DMA,
        pltpu.SemaphoreType.DMA,
    ],
)
def _sc_agg(src_hbm, dst_hbm, h_hbm, out_hbm, src_v, dst_v, rows_v, zbuf,
            agg_sh, s0, s1, s2, s3, s4):
    c = lax.axis_index("c")
    s = lax.axis_index("s")
    wid = s * NC + c
    sems = [s0, s1, s2, s3, s4]

    def _zrow(i, carry):
        for j in range(H // 16):
            zbuf[i, pl.ds(j * 16, 16)] = jnp.zeros((16,), jnp.float32)
        return carry
    lax.fori_loop(0, ZROWS, _zrow, 0)
    for t in range(ROWS_PER_SUB // ZROWS):
        pltpu.sync_copy(
            zbuf, agg_sh.at[pl.ds(s * ROWS_PER_SUB + t * ZROWS, ZROWS)])
    plsc.subcore_barrier()

    pltpu.sync_copy(src_hbm.at[wid], src_v)
    pltpu.sync_copy(dst_hbm.at[wid], dst_v)

    for r in range(RING):
        pltpu.async_copy(h_hbm.at[src_v.at[r]], rows_v.at[r], sems[r])

    def _body(gi, carry):
        g = gi * RING
        for r in range(RING):
            k = g + r
            pltpu.make_async_copy(h_hbm.at[src_v.at[k]], rows_v.at[r],
                                  sems[r]).wait()
            pltpu.sync_copy(rows_v.at[r], agg_sh.at[dst_v.at[k]], add=True)
            pltpu.async_copy(h_hbm.at[src_v.at[k + RING]], rows_v.at[r],
                             sems[r])
        return carry
    lax.fori_loop(0, NCHUNK // RING - 1, _body, 0)

    g = NCHUNK - RING
    for r in range(RING):
        k = g + r
        pltpu.make_async_copy(h_hbm.at[src_v.at[k]], rows_v.at[r],
                              sems[r]).wait()
        pltpu.sync_copy(rows_v.at[r], agg_sh.at[dst_v.at[k]], add=True)

    plsc.subcore_barrier()
    pltpu.sync_copy(agg_sh.at[pl.ds(s * ROWS_PER_SUB, ROWS_PER_SUB)],
                    out_hbm.at[c, pl.ds(s * ROWS_PER_SUB, ROWS_PER_SUB)])


@functools.partial(
    pl.kernel,
    out_type=jax.ShapeDtypeStruct((NPAD, H), jnp.float32),
    mesh=_MESH,
    scratch_types=[
        pltpu.VMEM((EMB_CHUNKS, CHUNK), jnp.int32),
        pltpu.VMEM((CHUNK, H), jnp.float32),
        pltpu.SemaphoreType.DMA,
    ],
)
def _sc_embed(state_hbm, emb_hbm, out_hbm, idx_v, rows_v, sem):
    c = lax.axis_index("c")
    s = lax.axis_index("s")
    wid = s * NC + c
    pltpu.sync_copy(state_hbm.at[wid], idx_v)
    for k in range(EMB_CHUNKS):
        pltpu.async_copy(emb_hbm.at[idx_v.at[k]], rows_v, sem).wait()
        pltpu.sync_copy(
            rows_v, out_hbm.at[pl.ds(wid * EMB_PER_W + k * CHUNK, CHUNK)])


def _mlp_body(h_ref, a0_ref, a1_ref, w1_ref, w2_ref, p_ref, out_ref):
    z = h_ref[...] + a0_ref[...] + a1_ref[...]
    u = jnp.dot(z, w1_ref[...], preferred_element_type=jnp.float32)
    mu = jnp.mean(u, axis=0, keepdims=True)
    var = jnp.mean(jnp.square(u - mu), axis=0, keepdims=True)
    u = (u - mu) * lax.rsqrt(var + BN_EPS) * p_ref[0:1, :] + p_ref[1:2, :]
    u = jnp.maximum(u, 0.0)
    v = jnp.dot(u, w2_ref[...], preferred_element_type=jnp.float32)
    mu2 = jnp.mean(v, axis=0, keepdims=True)
    var2 = jnp.mean(jnp.square(v - mu2), axis=0, keepdims=True)
    v = (v - mu2) * lax.rsqrt(var2 + BN_EPS) * p_ref[2:3, :] + p_ref[3:4, :]
    out_ref[...] = jnp.maximum(v, 0.0)


_tc_mlp = pl.pallas_call(
    _mlp_body,
    out_shape=jax.ShapeDtypeStruct((N, H), jnp.float32),
)


def _readout_body(h0, h1, h2, h3, h4, wr1_ref, wr2_ref, b_ref, out_ref):
    hs = (h0, h1, h2, h3, h4)
    acc = b_ref[0:1, :]
    for i in range(L):
        acc = acc + jnp.dot(hs[i][...], wr1_ref[i],
                            preferred_element_type=jnp.float32)
    y = jnp.maximum(acc, 0.0)
    out_ref[...] = (
        jnp.dot(y, wr2_ref[...], preferred_element_type=jnp.float32)
        + b_ref[1:2, :])


_tc_readout = pl.pallas_call(
    _readout_body,
    out_shape=jax.ShapeDtypeStruct((N, H), jnp.float32),
)


def kernel(state, edge_index, emb, W1, W2, g1, b1, g2, b2, Wr1, br1, Wr2, br2):
    src = edge_index[0].astype(jnp.int32).reshape(NW, NCHUNK, CHUNK)
    dst = edge_index[1].astype(jnp.int32).reshape(NW, NCHUNK, CHUNK)
    state_p = jnp.pad(state.astype(jnp.int32), (0, NPAD - N))
    state_p = state_p.reshape(NW, EMB_CHUNKS, CHUNK)

    h = _sc_embed(state_p, emb)[:N]
    hs = [h]
    for i in range(L - 1):
        parts = _sc_agg(src, dst, h)
        p = jnp.stack([g1[i], b1[i], g2[i], b2[i]], axis=0)
        p = jnp.concatenate([p, jnp.zeros((4, H), jnp.float32)], axis=0)
        h = _tc_mlp(h, parts[0], parts[1], W1[i], W2[i], p)
        hs.append(h)

    wr1 = Wr1.reshape(L, H, H)
    wr2 = jnp.zeros((H, H), jnp.float32).at[:, :OUT].set(Wr2)
    b = jnp.zeros((8, H), jnp.float32)
    b = b.at[0, :].set(br1)
    b = b.at[1, :OUT].set(br2)
    score = _tc_readout(hs[0], hs[1], hs[2], hs[3], hs[4], wr1, wr2, b)
    return score[:, :OUT]


# speedup vs baseline: 8.6296x; 8.6296x over previous
import functools

import jax
import jax.numpy as jnp
from jax import lax
from jax.experimental import pallas as pl
from jax.experimental.pallas import tpu as pltpu
from jax.experimental.pallas import tpu_sc as plsc

N = 10000
E = 320000
H = 128
L = 5
OUT = 2
BN_EPS = 1e-5

NC = 2
NS = 16
NW = NC * NS
HH = H // NC
EDGES_PER_T = E // NS
CHUNK = 80
NCHUNK = EDGES_PER_T // CHUNK
RING = 5
NPAD = 10240
ZROWS = 128
ROWS_PER_SUB = NPAD // NS
EMB_PER_W = NPAD // NW
EMB_CHUNKS = EMB_PER_W // CHUNK

def _sc_agg_body(src_hbm, dst_hbm, h2_hbm, out_hbm, src_v, dst_v, rows_v, zbuf,
                 agg_sh, s0, s1, s2, s3, s4):
    c = lax.axis_index("c")
    s = lax.axis_index("s")
    sems = [s0, s1, s2, s3, s4]

    def _zrow(i, carry):
        for j in range(HH // 16):
            zbuf[i, pl.ds(j * 16, 16)] = jnp.zeros((16,), jnp.float32)
        return carry
    lax.fori_loop(0, ZROWS, _zrow, 0)
    for t in range(ROWS_PER_SUB // ZROWS):
        pltpu.sync_copy(
            zbuf, agg_sh.at[pl.ds(s * ROWS_PER_SUB + t * ZROWS, ZROWS)])
    plsc.subcore_barrier()

    pltpu.sync_copy(src_hbm.at[s], src_v)
    pltpu.sync_copy(dst_hbm.at[s], dst_v)

    def _sxform(i, carry):
        for j in range(CHUNK // 16):
            v = src_v[i, pl.ds(j * 16, 16)]
            src_v[i, pl.ds(j * 16, 16)] = v * 2 + c
        return carry
    lax.fori_loop(0, NCHUNK, _sxform, 0)

    for r in range(RING):
        pltpu.async_copy(h2_hbm.at[src_v.at[r]], rows_v.at[r], sems[r])

    def _body(gi, carry):
        g = gi * RING
        for r in range(RING):
            k = g + r
            pltpu.make_async_copy(h2_hbm.at[src_v.at[k]], rows_v.at[r],
                                  sems[r]).wait()
            pltpu.sync_copy(rows_v.at[r], agg_sh.at[dst_v.at[k]], add=True)
            pltpu.async_copy(h2_hbm.at[src_v.at[k + RING]], rows_v.at[r],
                             sems[r])
        return carry
    lax.fori_loop(0, NCHUNK // RING - 1, _body, 0)

    g = NCHUNK - RING
    for r in range(RING):
        k = g + r
        pltpu.make_async_copy(h2_hbm.at[src_v.at[k]], rows_v.at[r],
                              sems[r]).wait()
        pltpu.sync_copy(rows_v.at[r], agg_sh.at[dst_v.at[k]], add=True)

    plsc.subcore_barrier()
    pltpu.sync_copy(agg_sh.at[pl.ds(s * ROWS_PER_SUB, ROWS_PER_SUB)],
                    out_hbm.at[pl.ds(s * ROWS_PER_SUB, ROWS_PER_SUB), c])


def _sc_embed_body(state_hbm, emb_hbm, out_hbm, idx_v, rows_v, sem):
    c = lax.axis_index("c")
    s = lax.axis_index("s")
    wid = s * NC + c
    pltpu.sync_copy(state_hbm.at[wid], idx_v)
    for k in range(EMB_CHUNKS):
        pltpu.async_copy(emb_hbm.at[idx_v.at[k]], rows_v, sem).wait()
        pltpu.sync_copy(
            rows_v, out_hbm.at[pl.ds(wid * EMB_PER_W + k * CHUNK, CHUNK)])


@functools.cache
def _sc_kernels():
    mesh = plsc.VectorSubcoreMesh(core_axis_name="c", subcore_axis_name="s",
                                  num_cores=NC, num_subcores=NS)
    agg = pl.kernel(
        _sc_agg_body,
        out_type=jax.ShapeDtypeStruct((NPAD, NC, HH), jnp.float32),
        mesh=mesh,
        compiler_params=pltpu.CompilerParams(use_tc_tiling_on_sc=False),
        scratch_types=[
            pltpu.VMEM((NCHUNK, CHUNK), jnp.int32),
            pltpu.VMEM((NCHUNK, CHUNK), jnp.int32),
            pltpu.VMEM((RING, CHUNK, HH), jnp.float32),
            pltpu.VMEM((ZROWS, HH), jnp.float32),
            pltpu.VMEM_SHARED((NPAD, HH), jnp.float32),
            pltpu.SemaphoreType.DMA,
            pltpu.SemaphoreType.DMA,
            pltpu.SemaphoreType.DMA,
            pltpu.SemaphoreType.DMA,
            pltpu.SemaphoreType.DMA,
        ],
    )
    embed = pl.kernel(
        _sc_embed_body,
        out_type=jax.ShapeDtypeStruct((NPAD, H), jnp.float32),
        mesh=mesh,
        scratch_types=[
            pltpu.VMEM((EMB_CHUNKS, CHUNK), jnp.int32),
            pltpu.VMEM((CHUNK, H), jnp.float32),
            pltpu.SemaphoreType.DMA,
        ],
    )
    return agg, embed


def _mlp_body(h_ref, a_ref, w1_ref, w2_ref, p_ref, out_ref):
    z = h_ref[...] + a_ref[...]
    u = jnp.dot(z, w1_ref[...], preferred_element_type=jnp.float32)
    mu = jnp.mean(u, axis=0, keepdims=True)
    var = jnp.mean(jnp.square(u - mu), axis=0, keepdims=True)
    u = (u - mu) / jnp.sqrt(var + BN_EPS) * p_ref[0:1, :] + p_ref[1:2, :]
    u = jnp.maximum(u, 0.0)
    v = jnp.dot(u, w2_ref[...], preferred_element_type=jnp.float32)
    mu2 = jnp.mean(v, axis=0, keepdims=True)
    var2 = jnp.mean(jnp.square(v - mu2), axis=0, keepdims=True)
    v = (v - mu2) / jnp.sqrt(var2 + BN_EPS) * p_ref[2:3, :] + p_ref[3:4, :]
    out_ref[...] = jnp.maximum(v, 0.0)


_tc_mlp = pl.pallas_call(
    _mlp_body,
    out_shape=jax.ShapeDtypeStruct((N, H), jnp.float32),
)


def _readout_body(h0, h1, h2, h3, h4, wr1_ref, wr2_ref, b_ref, out_ref):
    hs = (h0, h1, h2, h3, h4)
    acc = b_ref[0:1, :]
    for i in range(L):
        acc = acc + jnp.dot(hs[i][...], wr1_ref[i],
                            preferred_element_type=jnp.float32)
    y = jnp.maximum(acc, 0.0)
    out_ref[...] = (
        jnp.dot(y, wr2_ref[...], preferred_element_type=jnp.float32)
        + b_ref[1:2, :])


_RB = 2000
_tc_readout = pl.pallas_call(
    _readout_body,
    grid=(N // _RB,),
    in_specs=[pl.BlockSpec((_RB, H), lambda i: (i, 0))] * L
    + [
        pl.BlockSpec((L, H, H), lambda i: (0, 0, 0)),
        pl.BlockSpec((H, H), lambda i: (0, 0)),
        pl.BlockSpec((8, H), lambda i: (0, 0)),
    ],
    out_specs=pl.BlockSpec((_RB, H), lambda i: (i, 0)),
    out_shape=jax.ShapeDtypeStruct((N, H), jnp.float32),
)


def kernel(state, edge_index, emb, W1, W2, g1, b1, g2, b2, Wr1, br1, Wr2, br2):
    src = edge_index[0].astype(jnp.int32).reshape(NS, NCHUNK, CHUNK)
    dst = edge_index[1].astype(jnp.int32).reshape(NS, NCHUNK, CHUNK)
    state_p = jnp.pad(state.astype(jnp.int32), (0, NPAD - N))
    state_p = state_p.reshape(NW, EMB_CHUNKS, CHUNK)

    sc_agg, sc_embed = _sc_kernels()
    h = sc_embed(state_p, emb)[:N]
    hs = [h]
    for i in range(L - 1):
        h2 = h.reshape(2 * N, HH)
        agg = sc_agg(src, dst, h2).reshape(NPAD, H)[:N]
        p = jnp.stack([g1[i], b1[i], g2[i], b2[i]], axis=0)
        p = jnp.concatenate([p, jnp.zeros((4, H), jnp.float32)], axis=0)
        h = _tc_mlp(h, agg, W1[i], W2[i], p)
        hs.append(h)

    wr1 = Wr1.reshape(L, H, H)
    wr2 = jnp.zeros((H, H), jnp.float32).at[:, :OUT].set(Wr2)
    b = jnp.zeros((8, H), jnp.float32)
    b = b.at[0, :].set(br1)
    b = b.at[1, :OUT].set(br2)
    score = _tc_readout(hs[0], hs[1], hs[2], hs[3], hs[4], wr1, wr2, b)
    return score[:, :OUT]
